# baseline (speedup 1.0000x reference)
import functools

import jax
import jax.numpy as jnp
from jax.experimental import pallas as pl
from jax.experimental.pallas import tpu as pltpu


def _focal_kernel(o_ref, t_ref, out_ref, *, gamma, inv_n):
    diff = o_ref[...] - t_ref[...]
    sq = diff * diff
    row_loss = jnp.sum(sq, axis=1, keepdims=True)
    w = 1.0 - jnp.exp(-row_loss)
    wg = w
    for _ in range(int(gamma) - 1):
        wg = wg * w
    focal = wg * row_loss
    s = jnp.sum(focal, axis=2, keepdims=True)
    part = jnp.sum(s, axis=0, keepdims=True) * inv_n

    j = pl.program_id(1)

    @pl.when(j == 0)
    def _init():
        out_ref[...] = part

    @pl.when(j != 0)
    def _acc():
        out_ref[...] += part


def kernel(outputs, targets):
    gamma = 2
    B, S, D = outputs.shape
    n_items = B * S

    o_t = outputs.transpose(0, 2, 1)
    t_t = targets.transpose(0, 2, 1)

    bb = 32
    while B % bb != 0:
        bb //= 2
    grid = B // bb

    n_cores = 2 if grid % 2 == 0 else 1
    g2 = grid // n_cores

    kern = functools.partial(_focal_kernel, gamma=gamma,
                             inv_n=1.0 / float(n_items))
    partials = pl.pallas_call(
        kern,
        out_shape=jax.ShapeDtypeStruct((n_cores, 1, 1), jnp.float32),
        grid_spec=pltpu.PrefetchScalarGridSpec(
            num_scalar_prefetch=0,
            grid=(n_cores, g2),
            in_specs=[
                pl.BlockSpec((bb, D, S), lambda c, j: (c * g2 + j, 0, 0)),
                pl.BlockSpec((bb, D, S), lambda c, j: (c * g2 + j, 0, 0)),
            ],
            out_specs=pl.BlockSpec((1, 1, 1), lambda c, j: (c, 0, 0)),
        ),
        compiler_params=pltpu.CompilerParams(
            dimension_semantics=("parallel", "arbitrary"),
            vmem_limit_bytes=64 * 1024 * 1024,
        ),
    )(o_t, t_t)
    total = partials[0, 0, 0]
    for c in range(1, n_cores):
        total = total + partials[c, 0, 0]
    return total


# speedup vs baseline: 1.0992x; 1.0992x over previous
import functools

import jax
import jax.numpy as jnp
from jax.experimental import pallas as pl
from jax.experimental.pallas import tpu as pltpu


def _focal_kernel(o_ref, t_ref, out_ref, *, gamma, inv_n):
    diff = o_ref[...] - t_ref[...]
    sq = diff * diff
    row_loss = jnp.sum(sq, axis=1, keepdims=True)
    w = 1.0 - jnp.exp(-row_loss)
    wg = w
    for _ in range(int(gamma) - 1):
        wg = wg * w
    focal = wg * row_loss
    s = jnp.sum(focal, axis=2, keepdims=True)
    part = jnp.sum(s, axis=0, keepdims=True) * inv_n

    j = pl.program_id(1)

    @pl.when(j == 0)
    def _init():
        out_ref[...] = part

    @pl.when(j != 0)
    def _acc():
        out_ref[...] += part


def kernel(outputs, targets):
    gamma = 2
    B, S, D = outputs.shape
    n_items = B * S

    o_t = outputs.transpose(0, 2, 1)
    t_t = targets.transpose(0, 2, 1)

    bb = 32
    while B % bb != 0:
        bb //= 2
    grid = B // bb

    n_cores = 2 if grid % 2 == 0 else 1
    g2 = grid // n_cores

    kern = functools.partial(_focal_kernel, gamma=gamma,
                             inv_n=1.0 / float(n_items))
    partials = pl.pallas_call(
        kern,
        out_shape=jax.ShapeDtypeStruct((n_cores, 1, 1), jnp.float32),
        grid_spec=pltpu.PrefetchScalarGridSpec(
            num_scalar_prefetch=0,
            grid=(n_cores, g2),
            in_specs=[
                pl.BlockSpec((bb, D, S), lambda c, j: (c * g2 + j, 0, 0)),
                pl.BlockSpec((bb, D, S), lambda c, j: (c * g2 + j, 0, 0)),
            ],
            out_specs=pl.BlockSpec((1, 1, 1), lambda c, j: (c, 0, 0)),
        ),
        compiler_params=pltpu.CompilerParams(
            dimension_semantics=("parallel", "arbitrary"),
            vmem_limit_bytes=64 * 1024 * 1024,
        ),
    )(o_t, t_t)
    return jnp.sum(partials)
